# baseline (speedup 1.0000x reference)
import functools

import jax
import jax.numpy as jnp
from jax import lax
from jax.experimental import pallas as pl
from jax.experimental.pallas import tpu as pltpu
from jax.experimental.pallas import tpu_sc as plsc

FRAME_RATE = 12.5
F = 1500
N_ROWS = 32 * 448
NW = 32
RW = N_ROWS // NW
G = 16
NGROUPS = RW // G
UNROLL = 6
MAGIC = 0x5F3759DF


def _rsqrt_newton(x):
    i = plsc.bitcast(x, jnp.int32)
    y = plsc.bitcast(MAGIC - lax.shift_right_logical(i, 1), jnp.float32)
    for _ in range(3):
        y = y * (1.5 - 0.5 * x * y * y)
    return y


def _sumsq_ramp(n):
    return n * (n + 1.0) * (2.0 * n + 1.0) * (1.0 / 6.0)


def _sc_body(pred_hbm, ts_hbm, mask_hbm, num_hbm, den_hbm,
             tsbuf, maskbuf, buf0, buf1, numbuf, denbuf,
             sem0, sem1):
    c = lax.axis_index("c")
    s = lax.axis_index("s")
    wid = s * 2 + c
    row0 = wid * RW

    lane = lax.broadcasted_iota(jnp.int32, (16,), 0)
    lane_f = lane.astype(jnp.float32)
    idx0 = lane * F

    pltpu.sync_copy(ts_hbm.at[pl.ds(row0 * 2, RW * 2)], tsbuf)
    pltpu.sync_copy(mask_hbm.at[pl.ds(row0, RW)], maskbuf)

    def dma_start(g, buf, sem):
        return pltpu.async_copy(
            pred_hbm.at[pl.ds((row0 + g * G) * F, G * F)], buf, sem)

    def dma_wait(buf, sem):
        pltpu.make_async_copy(
            pred_hbm.at[pl.ds(0, G * F)], buf, sem).wait()

    dma_start(0, buf0, sem0)
    dma_start(1, buf1, sem1)

    def process_group(g, buf, num_acc, den_acc):
        gbase = g * G
        tidx = 2 * gbase + 2 * lane
        sv = plsc.load_gather(tsbuf, [tidx])
        ev = plsc.load_gather(tsbuf, [tidx + 1])
        sf = (sv * FRAME_RATE).astype(jnp.int32).astype(jnp.float32)
        sf = jnp.minimum(jnp.maximum(sf, 0.0), float(F - 1))
        ef = (ev * FRAME_RATE).astype(jnp.int32).astype(jnp.float32)
        ef = jnp.maximum(sf + 1.0, jnp.minimum(ef + 1.0, float(F)))

        n1 = jnp.minimum(jnp.maximum(4.0 - sf, 0.0), 4.0)
        n2 = jnp.minimum(jnp.maximum(ef - (F - 4.0), 0.0), 4.0)
        wsq = 25.0 * (ef - sf) + 60.0 - _sumsq_ramp(n1) - _sumsq_ramp(n2)

        def fbody(_, carry):
            idxv, rise, fall, dots, psqs = carry
            new_dots = []
            new_psqs = []
            for u in range(UNROLL):
                p = plsc.load_gather(buf, [idxv + u])
                w = jnp.minimum(jnp.minimum(rise + float(u), fall - float(u)),
                                5.0)
                w = jnp.maximum(w, 0.0)
                new_dots.append(dots[u] + w * p)
                new_psqs.append(psqs[u] + p * p)
            return (idxv + UNROLL, rise + float(UNROLL), fall - float(UNROLL),
                    tuple(new_dots), tuple(new_psqs))

        zero = jnp.zeros((16,), jnp.float32)
        zeros = (zero,) * UNROLL
        carry = (idx0, 5.0 - sf, ef + 4.0, zeros, zeros)
        carry = lax.fori_loop(0, F // UNROLL, fbody, carry)
        _, _, _, dots, psqs = carry
        dot = functools.reduce(lambda a, b: a + b, dots)
        psq = functools.reduce(lambda a, b: a + b, psqs)

        inv_pn = _rsqrt_newton(jnp.maximum(psq, 1e-16))
        inv_gn = _rsqrt_newton(0.04 * wsq)
        cos = (0.2 * dot) * inv_pn * inv_gn

        mv = maskbuf[pl.ds(gbase, 16)]
        return num_acc + (1.0 - cos) * mv, den_acc + mv

    def outer(k, carry):
        num_acc, den_acc = carry
        g0 = 2 * k
        dma_wait(buf0, sem0)
        num_acc, den_acc = process_group(g0, buf0, num_acc, den_acc)

        @pl.when(g0 + 2 < NGROUPS)
        def _():
            dma_start(g0 + 2, buf0, sem0)

        dma_wait(buf1, sem1)
        num_acc, den_acc = process_group(g0 + 1, buf1, num_acc, den_acc)

        @pl.when(g0 + 3 < NGROUPS)
        def _():
            dma_start(g0 + 3, buf1, sem1)

        return num_acc, den_acc

    zero = jnp.zeros((16,), jnp.float32)
    num_acc, den_acc = lax.fori_loop(0, NGROUPS // 2, outer, (zero, zero))

    numbuf[...] = num_acc
    denbuf[...] = den_acc
    pltpu.sync_copy(numbuf, num_hbm.at[pl.ds(wid * 16, 16)])
    pltpu.sync_copy(denbuf, den_hbm.at[pl.ds(wid * 16, 16)])


def _final_body(num_ref, den_ref, out_ref):
    num = jnp.sum(num_ref[...])
    den = jnp.sum(den_ref[...])
    out_ref[0, 0] = num / jnp.maximum(den, 1.0)


def kernel(predicted_attn, token_timestamps, attention_mask):
    B, T, Fdim = predicted_attn.shape
    pred = predicted_attn.reshape(B * T * Fdim)
    ts = token_timestamps.reshape(B * T * 2)
    mask = attention_mask.astype(jnp.float32).reshape(B * T)

    mesh = plsc.VectorSubcoreMesh(core_axis_name="c", subcore_axis_name="s")
    sc = functools.partial(
        pl.kernel,
        mesh=mesh,
        compiler_params=pltpu.CompilerParams(needs_layout_passes=False),
        out_type=(
            jax.ShapeDtypeStruct((NW * 16,), jnp.float32),
            jax.ShapeDtypeStruct((NW * 16,), jnp.float32),
        ),
        scratch_types=[
            pltpu.VMEM((RW * 2,), jnp.float32),
            pltpu.VMEM((RW,), jnp.float32),
            pltpu.VMEM((G * F,), jnp.float32),
            pltpu.VMEM((G * F,), jnp.float32),
            pltpu.VMEM((16,), jnp.float32),
            pltpu.VMEM((16,), jnp.float32),
            pltpu.SemaphoreType.DMA,
            pltpu.SemaphoreType.DMA,
        ],
    )(_sc_body)
    num, den = sc(pred, ts, mask)

    out = pl.pallas_call(
        _final_body,
        grid=(1,),
        in_specs=[
            pl.BlockSpec((4, 128), lambda i: (0, 0)),
            pl.BlockSpec((4, 128), lambda i: (0, 0)),
        ],
        out_specs=pl.BlockSpec(memory_space=pltpu.SMEM),
        out_shape=jax.ShapeDtypeStruct((1, 1), jnp.float32),
    )(num.reshape(4, 128), den.reshape(4, 128))
    return out[0, 0]


# speedup vs baseline: 1.8304x; 1.8304x over previous
import functools

import jax
import jax.numpy as jnp
from jax import lax
from jax.experimental import pallas as pl
from jax.experimental.pallas import tpu as pltpu
from jax.experimental.pallas import tpu_sc as plsc

FRAME_RATE = 12.5
F = 1500
N_ROWS = 32 * 448
NW = 32
RW = N_ROWS // NW
G = 16
NGROUPS = RW // G
NFULL = 93
RUN = 3
MAGIC = 0x5F3759DF


def _rsqrt_newton(x):
    i = plsc.bitcast(x, jnp.int32)
    y = plsc.bitcast(MAGIC - lax.shift_right_logical(i, 1), jnp.float32)
    for _ in range(3):
        y = y * (1.5 - 0.5 * x * y * y)
    return y


def _sumsq_ramp(n):
    return n * (n + 1.0) * (2.0 * n + 1.0) * (1.0 / 6.0)


def _sc_body(pred_hbm, ts_hbm, mask_hbm, num_hbm, den_hbm,
             tsbuf, maskbuf, buf0, buf1, numbuf, denbuf,
             sem0, sem1):
    c = lax.axis_index("c")
    s = lax.axis_index("s")
    wid = s * 2 + c
    row0 = wid * RW

    lane = lax.broadcasted_iota(jnp.int32, (16,), 0)
    lane_f = lane.astype(jnp.float32)

    pltpu.sync_copy(ts_hbm.at[pl.ds(row0 * 2, RW * 2)], tsbuf)
    pltpu.sync_copy(mask_hbm.at[pl.ds(row0, RW)], maskbuf)

    def dma_start(g, buf, sem):
        return pltpu.async_copy(
            pred_hbm.at[pl.ds(row0 + g * G, G), :], buf, sem)

    def dma_wait(buf, sem):
        pltpu.make_async_copy(
            pred_hbm.at[pl.ds(0, G), :], buf, sem).wait()

    dma_start(0, buf0, sem0)
    dma_start(1, buf1, sem1)

    def process_group(g, buf, num_acc, den_acc):
        gbase = g * G
        tidx = 2 * gbase + 2 * lane
        sv = plsc.load_gather(tsbuf, [tidx])
        ev = plsc.load_gather(tsbuf, [tidx + 1])
        sf = (sv * FRAME_RATE).astype(jnp.int32).astype(jnp.float32)
        sf = jnp.minimum(jnp.maximum(sf, 0.0), float(F - 1))
        ef = (ev * FRAME_RATE).astype(jnp.int32).astype(jnp.float32)
        ef = jnp.maximum(sf + 1.0, jnp.minimum(ef + 1.0, float(F)))

        n1 = jnp.minimum(jnp.maximum(4.0 - sf, 0.0), 4.0)
        n2 = jnp.minimum(jnp.maximum(ef - (F - 4.0), 0.0), 4.0)
        wsq = 25.0 * (ef - sf) + 60.0 - _sumsq_ramp(n1) - _sumsq_ramp(n2)

        av = 5.0 - sf
        bv = ef + 4.0

        zero = jnp.zeros((16,), jnp.float32)
        dotv = zero
        psqv = zero
        for r in range(G):
            rise0 = lane_f + av[r]
            fall0 = bv[r] - lane_f

            def fbody(j, carry, _r=r):
                rise, fall, accs = carry
                off = j * (16 * RUN)
                new = []
                for u in range(RUN):
                    p = buf[_r, pl.ds(off + u * 16, 16)]
                    w = jnp.minimum(
                        jnp.minimum(rise + float(16 * u),
                                    fall - float(16 * u)), 5.0)
                    w = jnp.maximum(w, 0.0)
                    d, q = accs[u]
                    new.append((d + w * p, q + p * p))
                return (rise + float(16 * RUN), fall - float(16 * RUN),
                        tuple(new))

            accs0 = ((zero, zero),) * RUN
            rise_t, fall_t, accs = lax.fori_loop(
                0, NFULL // RUN, fbody, (rise0, fall0, accs0))

            p = buf[r, pl.ds(F - 16, 16)]
            p = jnp.where(lane >= 4, p, 0.0)
            w = jnp.minimum(jnp.minimum(rise_t - 4.0, fall_t + 4.0), 5.0)
            w = jnp.maximum(w, 0.0)
            (d0, q0), (d1, q1), (d2, q2) = accs
            d0 = d0 + w * p
            q0 = q0 + p * p
            dot_r = jnp.sum((d0 + d1) + d2)
            psq_r = jnp.sum((q0 + q1) + q2)
            dotv = jnp.where(lane == r, dot_r, dotv)
            psqv = jnp.where(lane == r, psq_r, psqv)

        inv_pn = _rsqrt_newton(jnp.maximum(psqv, 1e-16))
        inv_gn = _rsqrt_newton(0.04 * wsq)
        cos = (0.2 * dotv) * inv_pn * inv_gn

        mv = maskbuf[pl.ds(gbase, 16)]
        return num_acc + (1.0 - cos) * mv, den_acc + mv

    def outer(k, carry):
        num_acc, den_acc = carry
        g0 = 2 * k
        dma_wait(buf0, sem0)
        num_acc, den_acc = process_group(g0, buf0, num_acc, den_acc)

        @pl.when(g0 + 2 < NGROUPS)
        def _():
            dma_start(g0 + 2, buf0, sem0)

        dma_wait(buf1, sem1)
        num_acc, den_acc = process_group(g0 + 1, buf1, num_acc, den_acc)

        @pl.when(g0 + 3 < NGROUPS)
        def _():
            dma_start(g0 + 3, buf1, sem1)

        return num_acc, den_acc

    zero = jnp.zeros((16,), jnp.float32)
    num_acc, den_acc = lax.fori_loop(0, NGROUPS // 2, outer, (zero, zero))

    numbuf[...] = num_acc
    denbuf[...] = den_acc
    pltpu.sync_copy(numbuf, num_hbm.at[pl.ds(wid * 16, 16)])
    pltpu.sync_copy(denbuf, den_hbm.at[pl.ds(wid * 16, 16)])


def _final_body(num_ref, den_ref, out_ref):
    num = jnp.sum(num_ref[...])
    den = jnp.sum(den_ref[...])
    out_ref[0, 0] = num / jnp.maximum(den, 1.0)


def kernel(predicted_attn, token_timestamps, attention_mask):
    B, T, Fdim = predicted_attn.shape
    pred = predicted_attn.reshape(B * T, Fdim)
    ts = token_timestamps.reshape(B * T * 2)
    mask = attention_mask.astype(jnp.float32).reshape(B * T)

    mesh = plsc.VectorSubcoreMesh(core_axis_name="c", subcore_axis_name="s")
    sc = functools.partial(
        pl.kernel,
        mesh=mesh,
        compiler_params=pltpu.CompilerParams(needs_layout_passes=False),
        out_type=(
            jax.ShapeDtypeStruct((NW * 16,), jnp.float32),
            jax.ShapeDtypeStruct((NW * 16,), jnp.float32),
        ),
        scratch_types=[
            pltpu.VMEM((RW * 2,), jnp.float32),
            pltpu.VMEM((RW,), jnp.float32),
            pltpu.VMEM((G, F), jnp.float32),
            pltpu.VMEM((G, F), jnp.float32),
            pltpu.VMEM((16,), jnp.float32),
            pltpu.VMEM((16,), jnp.float32),
            pltpu.SemaphoreType.DMA,
            pltpu.SemaphoreType.DMA,
        ],
    )(_sc_body)
    num, den = sc(pred, ts, mask)

    out = pl.pallas_call(
        _final_body,
        grid=(1,),
        in_specs=[
            pl.BlockSpec((4, 128), lambda i: (0, 0)),
            pl.BlockSpec((4, 128), lambda i: (0, 0)),
        ],
        out_specs=pl.BlockSpec(memory_space=pltpu.SMEM),
        out_shape=jax.ShapeDtypeStruct((1, 1), jnp.float32),
    )(num.reshape(4, 128), den.reshape(4, 128))
    return out[0, 0]
